# baseline (speedup 1.0000x reference)
import jax
import jax.numpy as jnp
from jax import lax
from jax.experimental import pallas as pl
from jax.experimental.pallas import tpu as pltpu
from jax.experimental.pallas import tpu_sc as plsc

NUM_CLASSES = 10000
N_ROWS = 320000
D_COLS = 128
NC = 2
NS = 16
BLK = 128
NBLK = N_ROWS // BLK
BLKS_PER_TILE = NBLK // NS
EXTRA = NBLK - BLKS_PER_TILE * NS
CPT = NUM_CLASSES // NS
HALF = D_COLS // NC
W = 2 * HALF + 16


def _sc_body(x_hbm, lab_hbm, numw_hbm, miu_hbm, std_hbm,
             acc_sh, pa, pb, idxb,
             isem_a, isem_b, ssem_a, ssem_b):
    cid = lax.axis_index("c")
    sid = lax.axis_index("s")
    c0 = cid * HALF
    bufs = ((pa, isem_a, ssem_a), (pb, isem_b, ssem_b))

    def xslice(b):
        return x_hbm.at[pl.ds(b * BLK, BLK), pl.ds(c0, HALF)]

    zeros16 = jnp.zeros((16,), jnp.float32)
    ones16 = jnp.ones((16,), jnp.float32)

    @pl.loop(0, BLK)
    def _(i):
        for c4 in range(W // 16):
            pa[i, pl.ds(c4 * 16, 16)] = zeros16

    base = sid * CPT
    for j in range(5):
        pltpu.sync_copy(pa.at[pl.ds(0, 125), :],
                        acc_sh.at[pl.ds(base + j * 125, 125), :])

    @pl.loop(0, BLK)
    def _(i):
        pa[i, pl.ds(2 * HALF, 16)] = ones16
        pb[i, pl.ds(2 * HALF, 16)] = ones16

    for par in range(2):
        pv, isem, _ = bufs[par]
        pltpu.async_copy(xslice(sid + par * NS),
                         pv.at[:, pl.ds(0, HALF)], isem)
        pltpu.async_copy(lab_hbm.at[sid + par * NS], idxb.at[par], isem)

    plsc.subcore_barrier()

    def square(pv):
        @pl.loop(0, BLK)
        def _(i):
            for c4 in range(HALF // 16):
                v = pv[i, pl.ds(c4 * 16, 16)]
                pv[i, pl.ds(HALF + c4 * 16, 16)] = v * v

    @pl.loop(0, BLKS_PER_TILE, step=2)
    def _(k):
        for par in range(2):
            kk = k + par
            pv, isem, ssem = bufs[par]
            idx = idxb.at[par]
            pltpu.make_async_copy(xslice(sid), pv.at[:, pl.ds(0, HALF)],
                                  isem).wait()
            pltpu.make_async_copy(lab_hbm.at[sid], idx, isem).wait()
            square(pv)
            cp = pltpu.async_copy(pv, acc_sh.at[idx], ssem, add=True)
            cp.wait()

            @pl.when(kk + 2 < BLKS_PER_TILE)
            def _():
                b_next = sid + (kk + 2) * NS
                pltpu.async_copy(xslice(b_next), pv.at[:, pl.ds(0, HALF)],
                                 isem)
                pltpu.async_copy(lab_hbm.at[b_next], idx, isem)

    @pl.when(sid < EXTRA)
    def _():
        b = BLKS_PER_TILE * NS + sid
        pv, _, _ = bufs[0]
        idx = idxb.at[0]
        pltpu.sync_copy(xslice(b), pv.at[:, pl.ds(0, HALF)])
        pltpu.sync_copy(lab_hbm.at[b], idx)
        square(pv)
        pltpu.sync_copy(pv, acc_sh.at[idx], add=True)

    plsc.subcore_barrier()

    rows = pl.ds(base, CPT)
    pltpu.sync_copy(acc_sh.at[rows, pl.ds(0, HALF)],
                    miu_hbm.at[rows, pl.ds(c0, HALF)])
    pltpu.sync_copy(acc_sh.at[rows, pl.ds(HALF, HALF)],
                    std_hbm.at[rows, pl.ds(c0, HALF)])

    @pl.when(cid == 0)
    def _():
        pltpu.sync_copy(acc_sh.at[rows, pl.ds(2 * HALF, 16)],
                        numw_hbm.at[rows, :])


@jax.jit
def _tracker(X, labels2d):
    mesh = plsc.VectorSubcoreMesh(core_axis_name="c", subcore_axis_name="s")
    f = pl.kernel(
        _sc_body,
        compiler_params=pltpu.CompilerParams(use_tc_tiling_on_sc=False),
        out_type=(
            jax.ShapeDtypeStruct((NUM_CLASSES, 16), jnp.float32),
            jax.ShapeDtypeStruct((NUM_CLASSES, D_COLS), jnp.float32),
            jax.ShapeDtypeStruct((NUM_CLASSES, D_COLS), jnp.float32),
        ),
        mesh=mesh,
        scratch_types=[
            pltpu.VMEM_SHARED((NUM_CLASSES, W), jnp.float32),
            pltpu.VMEM((BLK, W), jnp.float32),
            pltpu.VMEM((BLK, W), jnp.float32),
            pltpu.VMEM((2, 128), jnp.int32),
            pltpu.SemaphoreType.DMA,
            pltpu.SemaphoreType.DMA,
            pltpu.SemaphoreType.DMA,
            pltpu.SemaphoreType.DMA,
        ],
    )
    return f(X, labels2d)


def kernel(X, labels):
    labels2d = labels.astype(jnp.int32).reshape(N_ROWS // 128, 128)
    numw, miu, std = _tracker(X, labels2d)
    return (numw[:, :1], miu, std)


# speedup vs baseline: 1.1191x; 1.1191x over previous
import jax
import jax.numpy as jnp
from jax import lax
from jax.experimental import pallas as pl
from jax.experimental.pallas import tpu as pltpu
from jax.experimental.pallas import tpu_sc as plsc

NUM_CLASSES = 10000
N_ROWS = 320000
D_COLS = 128
NC = 2
NS = 16
BLK = 128
NBLK = N_ROWS // BLK
BLKS_PER_TILE = NBLK // NS
EXTRA = NBLK - BLKS_PER_TILE * NS
CPT = NUM_CLASSES // NS
HALF = D_COLS // NC
W = 2 * HALF + 16


def _sc_body(x_hbm, lab_hbm, numw_hbm, miu_hbm, std_hbm,
             acc_sh, pa, pb, idxb,
             isem_a, isem_b, ssem_a, ssem_b):
    cid = lax.axis_index("c")
    sid = lax.axis_index("s")
    c0 = cid * HALF
    bufs = ((pa, isem_a, ssem_a), (pb, isem_b, ssem_b))

    def xslice(b):
        return x_hbm.at[pl.ds(b * BLK, BLK), pl.ds(c0, HALF)]

    zeros16 = jnp.zeros((16,), jnp.float32)
    ones16 = jnp.ones((16,), jnp.float32)

    @pl.loop(0, BLK)
    def _(i):
        for c4 in range(W // 16):
            pa[i, pl.ds(c4 * 16, 16)] = zeros16

    base = sid * CPT
    for j in range(5):
        pltpu.sync_copy(pa.at[pl.ds(0, 125), :],
                        acc_sh.at[pl.ds(base + j * 125, 125), :])

    @pl.loop(0, BLK)
    def _(i):
        pa[i, pl.ds(2 * HALF, 16)] = ones16
        pb[i, pl.ds(2 * HALF, 16)] = ones16

    for par in range(2):
        pv, isem, _ = bufs[par]
        pltpu.async_copy(xslice(sid + par * NS),
                         pv.at[:, pl.ds(0, HALF)], isem)
        pltpu.async_copy(lab_hbm.at[sid + par * NS], idxb.at[par], isem)

    plsc.subcore_barrier()

    def square_range(pv, r0, r1):
        @pl.loop(r0, r1, step=4)
        def _(i):
            for r in range(4):
                for c4 in range(HALF // 16):
                    v = pv[i + r, pl.ds(c4 * 16, 16)]
                    pv[i + r, pl.ds(HALF + c4 * 16, 16)] = v * v

    def refill(par, b):
        pv, isem, _ = bufs[par]
        pltpu.async_copy(xslice(b), pv.at[:, pl.ds(0, HALF)], isem)
        pltpu.async_copy(lab_hbm.at[b], idxb.at[par], isem)

    def wait_in(par):
        pv, isem, _ = bufs[par]
        pltpu.make_async_copy(xslice(sid), pv.at[:, pl.ds(0, HALF)],
                              isem).wait()
        pltpu.make_async_copy(lab_hbm.at[sid], idxb.at[par], isem).wait()

    def drain_scatter(par):
        pv, _, ssem = bufs[par]
        pltpu.make_async_copy(pv, acc_sh.at[idxb.at[par]], ssem).wait()

    @pl.loop(0, BLKS_PER_TILE, step=2)
    def _(k):
        wait_in(0)
        square_range(pa, 0, BLK // 2)

        @pl.when(k >= 1)
        def _():
            drain_scatter(1)
            refill(1, sid + (k + 1) * NS)
        square_range(pa, BLK // 2, BLK)
        cp_a = pltpu.async_copy(pa, acc_sh.at[idxb.at[0]], ssem_a, add=True)

        wait_in(1)
        square_range(pb, 0, BLK // 2)
        cp_a.wait()

        @pl.when(k + 2 < BLKS_PER_TILE)
        def _():
            refill(0, sid + (k + 2) * NS)

        square_range(pb, BLK // 2, BLK)
        pltpu.async_copy(pb, acc_sh.at[idxb.at[1]], ssem_b, add=True)

    drain_scatter(1)

    @pl.when(sid < EXTRA)
    def _():
        b = BLKS_PER_TILE * NS + sid
        pv, _, _ = bufs[0]
        idx = idxb.at[0]
        pltpu.sync_copy(xslice(b), pv.at[:, pl.ds(0, HALF)])
        pltpu.sync_copy(lab_hbm.at[b], idx)
        square_range(pv, 0, BLK)
        pltpu.sync_copy(pv, acc_sh.at[idx], add=True)

    plsc.subcore_barrier()

    rows = pl.ds(base, CPT)
    pltpu.sync_copy(acc_sh.at[rows, pl.ds(0, HALF)],
                    miu_hbm.at[rows, pl.ds(c0, HALF)])
    pltpu.sync_copy(acc_sh.at[rows, pl.ds(HALF, HALF)],
                    std_hbm.at[rows, pl.ds(c0, HALF)])

    @pl.when(cid == 0)
    def _():
        pltpu.sync_copy(acc_sh.at[rows, pl.ds(2 * HALF, 16)],
                        numw_hbm.at[rows, :])


@jax.jit
def _tracker(X, labels2d):
    mesh = plsc.VectorSubcoreMesh(core_axis_name="c", subcore_axis_name="s")
    f = pl.kernel(
        _sc_body,
        compiler_params=pltpu.CompilerParams(use_tc_tiling_on_sc=False),
        out_type=(
            jax.ShapeDtypeStruct((NUM_CLASSES, 16), jnp.float32),
            jax.ShapeDtypeStruct((NUM_CLASSES, D_COLS), jnp.float32),
            jax.ShapeDtypeStruct((NUM_CLASSES, D_COLS), jnp.float32),
        ),
        mesh=mesh,
        scratch_types=[
            pltpu.VMEM_SHARED((NUM_CLASSES, W), jnp.float32),
            pltpu.VMEM((BLK, W), jnp.float32),
            pltpu.VMEM((BLK, W), jnp.float32),
            pltpu.VMEM((2, 128), jnp.int32),
            pltpu.SemaphoreType.DMA,
            pltpu.SemaphoreType.DMA,
            pltpu.SemaphoreType.DMA,
            pltpu.SemaphoreType.DMA,
        ],
    )
    return f(X, labels2d)


def kernel(X, labels):
    labels2d = labels.astype(jnp.int32).reshape(N_ROWS // 128, 128)
    numw, miu, std = _tracker(X, labels2d)
    return (numw[:, :1], miu, std)
